# Initial kernel scaffold; baseline (speedup 1.0000x reference)
import jax
import jax.numpy as jnp
from jax.experimental import pallas as pl


def kernel(x, table):
    raise NotImplementedError("write your pallas kernel here")


# speedup vs baseline: 1.2134x; 1.2134x over previous
import functools

import jax
import jax.numpy as jnp
from jax import lax
from jax.experimental import pallas as pl
from jax.experimental.pallas import tpu as pltpu
from jax.experimental.pallas import tpu_sc as plsc

_NUM_FIELDS = 26
_FIELD_SIZE = 100000
_BATCH = 16384


@functools.partial(jax.jit, static_argnames=())
def _fmlinear(x_t, tab):
    info = plsc.get_sparse_core_info()
    nw = info.num_cores * info.num_subcores
    lanes = info.num_lanes
    bw = _BATCH // nw

    mesh = plsc.VectorSubcoreMesh(core_axis_name="c", subcore_axis_name="s")

    @functools.partial(
        pl.kernel,
        mesh=mesh,
        out_type=jax.ShapeDtypeStruct((_BATCH,), jnp.float32),
        scratch_types=[
            pltpu.VMEM((bw,), jnp.int32),
            pltpu.VMEM((bw,), jnp.int32),
            pltpu.VMEM((bw,), jnp.float32),
            pltpu.VMEM((bw,), jnp.float32),
            pltpu.VMEM((bw,), jnp.float32),
            pltpu.SemaphoreType.DMA,
            pltpu.SemaphoreType.DMA,
        ],
    )
    def k(x_hbm, tab_hbm, out_hbm, idx0, idx1, val0, val1, acc, sem0, sem1):
        wid = lax.axis_index("s") * info.num_cores + lax.axis_index("c")
        base = wid * bw
        bufs = ((idx0, val0, sem0), (idx1, val1, sem1))

        def fire(f, idx_v, val_v, sem):
            pltpu.sync_copy(x_hbm.at[f, pl.ds(base, bw)], idx_v)
            off = f * _FIELD_SIZE
            if off:
                for i in range(bw // lanes):
                    s = pl.ds(i * lanes, lanes)
                    idx_v[s] = idx_v[s] + off
            return pltpu.async_copy(tab_hbm.at[idx_v], val_v, sem)

        cps = [fire(0, *bufs[0]), fire(1, *bufs[1])]
        for f in range(_NUM_FIELDS):
            p = f % 2
            idx_v, val_v, sem = bufs[p]
            cps[p].wait()
            for i in range(bw // lanes):
                s = pl.ds(i * lanes, lanes)
                if f == 0:
                    acc[s] = val_v[s]
                else:
                    acc[s] = acc[s] + val_v[s]
            if f + 2 < _NUM_FIELDS:
                cps[p] = fire(f + 2, idx_v, val_v, sem)

        pltpu.sync_copy(acc, out_hbm.at[pl.ds(base, bw)])

    return k(x_t, tab)


def kernel(x, table):
    x_t = x.T
    tab = table.reshape(-1)
    out = _fmlinear(x_t, tab)
    return out.reshape(_BATCH, 1)
